# Initial kernel scaffold; baseline (speedup 1.0000x reference)
import jax
import jax.numpy as jnp
from jax.experimental import pallas as pl


def kernel(test, question, tag, correct, mask, interaction, dffclt, dscrmn, gussng, testTag, user_correct_answer, user_total_answer, user_acc, user_mean, assessment_mean, test_mean, knowledgeTag_mean, time_to_solve, prior_testTag_frequency, W_interaction, W_test, W_question, W_tag, W_testTag, lin_w, lin_b, comb_w, comb_b):
    raise NotImplementedError("write your pallas kernel here")


# speedup vs baseline: 9.5349x; 9.5349x over previous
import functools

import jax
import jax.numpy as jnp
from jax import lax
from jax.experimental import pallas as pl
from jax.experimental.pallas import tpu as pltpu
from jax.experimental.pallas import tpu_sc as plsc

B, L = 4096, 200
R = B * L
D = 64
N_TESTS, N_QUESTIONS, N_TAGS = 1538, 9455, 913
NT1, NQ1, NG1 = N_TESTS + 1, N_QUESTIONS + 1, N_TAGS + 1
NCOMB = 3 * 10 * NG1

NW = 32
ROWS_PER_W = R // NW
CHUNK = 128
N_INNER = 8
N_OUTER = ROWS_PER_W // (CHUNK * N_INNER)
IDX_ROWS = R // CHUNK


def _idx_body(it_ref, tt_ref, tg_ref, out_ref):
    out_ref[...] = (it_ref[...] * 10 + tt_ref[...]) * NG1 + tg_ref[...]


def _compute_idxc(int2, tt2, tg2):
    blk = 128
    return pl.pallas_call(
        _idx_body,
        grid=(IDX_ROWS // blk,),
        in_specs=[pl.BlockSpec((blk, CHUNK), lambda i: (i, 0))] * 3,
        out_specs=pl.BlockSpec((blk, CHUNK), lambda i: (i, 0)),
        out_shape=jax.ShapeDtypeStruct((IDX_ROWS, CHUNK), jnp.int32),
    )(int2, tt2, tg2)


def _fold_body(wi, wt, wq, wg, wtt, cw, e4, et, eq):
    cw_ = cw[...]
    e_int = jnp.dot(wi[...], cw_[0:21], preferred_element_type=jnp.float32)
    e_tag = jnp.dot(wg[...], cw_[63:84], preferred_element_type=jnp.float32)
    e_tt = jnp.dot(wtt[...], cw_[99:104], preferred_element_type=jnp.float32)
    e4[...] = (e_int[:, None, None, :] + e_tt[None, :, None, :]
               + e_tag[None, None, :, :])
    et[...] = jnp.dot(wt[...], cw_[21:42], preferred_element_type=jnp.float32)
    eq[...] = jnp.dot(wq[...], cw_[42:63], preferred_element_type=jnp.float32)


def _fold_tables(wi, wt, wq, wg, wtt, cw):
    return pl.pallas_call(
        _fold_body,
        out_shape=(
            jax.ShapeDtypeStruct((3, 10, NG1, D), jnp.float32),
            jax.ShapeDtypeStruct((NT1, D), jnp.float32),
            jax.ShapeDtypeStruct((NQ1, D), jnp.float32),
        ),
    )(wi, wt, wq, wg, wtt, cw)


def _sc_gather_body(ecomb, etest, eq, idxc2, test2, q2, g_out,
                    ic, it, iq, acc, gb, gc, sem):
    wid = lax.axis_index("s") * 2 + lax.axis_index("c")
    row0 = wid * (ROWS_PER_W // CHUNK)

    def outer(o, carry_o):
        ro = row0 + o * N_INNER
        pltpu.sync_copy(idxc2.at[pl.ds(ro, N_INNER)], ic)
        pltpu.sync_copy(test2.at[pl.ds(ro, N_INNER)], it)
        pltpu.sync_copy(q2.at[pl.ds(ro, N_INNER)], iq)

        def inner(j, carry_i):
            r0 = (ro + j) * CHUNK
            c1 = pltpu.async_copy(ecomb.at[ic.at[j]], acc, sem)
            c2 = pltpu.async_copy(etest.at[it.at[j]], gb, sem)
            c3 = pltpu.async_copy(eq.at[iq.at[j]], gc, sem)
            c1.wait()
            c2.wait()
            c3.wait()

            def addrow(rr, carry_r):
                for cc in range(4):
                    s = pl.ds(cc * 16, 16)
                    acc[rr, s] = acc[rr, s] + gb[rr, s] + gc[rr, s]
                return carry_r

            lax.fori_loop(0, CHUNK, addrow, 0)
            pltpu.sync_copy(acc, g_out.at[pl.ds(r0, CHUNK)])
            return carry_i

        lax.fori_loop(0, N_INNER, inner, 0)
        return carry_o

    lax.fori_loop(0, N_OUTER, outer, 0)


def _sc_gather(ecomb, etest, eq, idxc2, test2, q2):
    mesh = plsc.VectorSubcoreMesh(core_axis_name="c", subcore_axis_name="s")
    run = functools.partial(
        pl.kernel,
        mesh=mesh,
        compiler_params=pltpu.CompilerParams(use_tc_tiling_on_sc=False),
        out_type=jax.ShapeDtypeStruct((R, D), jnp.float32),
        scratch_types=[
            pltpu.VMEM((N_INNER, CHUNK), jnp.int32),
            pltpu.VMEM((N_INNER, CHUNK), jnp.int32),
            pltpu.VMEM((N_INNER, CHUNK), jnp.int32),
            pltpu.VMEM((CHUNK, D), jnp.float32),
            pltpu.VMEM((CHUNK, D), jnp.float32),
            pltpu.VMEM((CHUNK, D), jnp.float32),
            pltpu.SemaphoreType.DMA,
        ],
    )(_sc_gather_body)
    return run(ecomb, etest, eq, idxc2, test2, q2)


def _final_body(f_ref, g_ref, wexp_ref, bexp_ref, csig_ref, bc_ref, x_ref):
    t = jnp.dot(f_ref[...], wexp_ref[...], preferred_element_type=jnp.float32)
    sg = jax.nn.sigmoid(t + bexp_ref[...])
    x_ref[...] = (jnp.dot(sg, csig_ref[...], preferred_element_type=jnp.float32)
                  + bc_ref[...] + g_ref[...])


def _final(f, g, wexp, bexp, csig, bc):
    br = 2048
    return pl.pallas_call(
        _final_body,
        grid=(R // br,),
        in_specs=[
            pl.BlockSpec((br, 12), lambda i: (i, 0)),
            pl.BlockSpec((br, D), lambda i: (i, 0)),
            pl.BlockSpec((12, D), lambda i: (0, 0)),
            pl.BlockSpec((1, D), lambda i: (0, 0)),
            pl.BlockSpec((D, D), lambda i: (0, 0)),
            pl.BlockSpec((1, D), lambda i: (0, 0)),
        ],
        out_specs=pl.BlockSpec((br, D), lambda i: (i, 0)),
        out_shape=jax.ShapeDtypeStruct((R, D), jnp.float32),
    )(f, g, wexp, bexp, csig, bc)


def kernel(test, question, tag, correct, mask, interaction, dffclt, dscrmn,
           gussng, testTag, user_correct_answer, user_total_answer, user_acc,
           user_mean, assessment_mean, test_mean, knowledgeTag_mean,
           time_to_solve, prior_testTag_frequency, W_interaction, W_test,
           W_question, W_tag, W_testTag, lin_w, lin_b, comb_w, comb_b):
    int2 = interaction.reshape(IDX_ROWS, CHUNK)
    tt2 = testTag.reshape(IDX_ROWS, CHUNK)
    tg2 = tag.reshape(IDX_ROWS, CHUNK)
    test2 = test.reshape(IDX_ROWS, CHUNK)
    q2 = question.reshape(IDX_ROWS, CHUNK)

    idxc2 = _compute_idxc(int2, tt2, tg2)
    e4, etest, eq = _fold_tables(W_interaction, W_test, W_question, W_tag,
                                 W_testTag, comb_w)
    ecomb = e4.reshape(NCOMB, D)
    g = _sc_gather(ecomb, etest, eq, idxc2, test2, q2)

    segs = [84, 89, 94] + [104 + 5 * i for i in range(9)]
    wexp = jnp.zeros((12, D), jnp.float32)
    bexp = jnp.zeros((1, D), jnp.float32)
    csig = jnp.zeros((D, D), jnp.float32)
    for k in range(12):
        wexp = wexp.at[k, 5 * k:5 * k + 5].set(lin_w)
        bexp = bexp.at[0, 5 * k:5 * k + 5].set(lin_b)
        csig = csig.at[5 * k:5 * k + 5, :].set(comb_w[segs[k]:segs[k] + 5, :])

    f = jnp.stack([dffclt, dscrmn, gussng, user_correct_answer,
                   user_total_answer, user_acc, user_mean, assessment_mean,
                   test_mean, knowledgeTag_mean, time_to_solve,
                   prior_testTag_frequency], axis=-1).reshape(R, 12)

    x = _final(f, g, wexp, bexp, csig, comb_b.reshape(1, D))
    return x.reshape(B, L, D)


# speedup vs baseline: 12.1409x; 1.2733x over previous
import functools

import jax
import jax.numpy as jnp
from jax import lax
from jax.experimental import pallas as pl
from jax.experimental.pallas import tpu as pltpu
from jax.experimental.pallas import tpu_sc as plsc

B, L = 4096, 200
R = B * L
D = 64
N_TESTS, N_QUESTIONS, N_TAGS = 1538, 9455, 913
NT1, NQ1, NG1 = N_TESTS + 1, N_QUESTIONS + 1, N_TAGS + 1
NCOMB = 3 * 10 * NG1

NW = 32
ROWS_PER_W = R // NW
CHUNK = 128
N_INNER = 8
N_OUTER = ROWS_PER_W // (CHUNK * N_INNER)
IDX_ROWS = R // CHUNK


def _idx_body(it_ref, tt_ref, tg_ref, out_ref):
    out_ref[...] = (it_ref[...] * 10 + tt_ref[...]) * NG1 + tg_ref[...]


def _compute_idxc(int2, tt2, tg2):
    blk = 128
    return pl.pallas_call(
        _idx_body,
        grid=(IDX_ROWS // blk,),
        in_specs=[pl.BlockSpec((blk, CHUNK), lambda i: (i, 0))] * 3,
        out_specs=pl.BlockSpec((blk, CHUNK), lambda i: (i, 0)),
        out_shape=jax.ShapeDtypeStruct((IDX_ROWS, CHUNK), jnp.int32),
    )(int2, tt2, tg2)


def _fold_body(wi, wt, wq, wg, wtt, cw, e4, et, eq):
    cw_ = cw[...]
    e_int = jnp.dot(wi[...], cw_[0:21], preferred_element_type=jnp.float32)
    e_tag = jnp.dot(wg[...], cw_[63:84], preferred_element_type=jnp.float32)
    e_tt = jnp.dot(wtt[...], cw_[99:104], preferred_element_type=jnp.float32)
    e4[...] = (e_int[:, None, None, :] + e_tt[None, :, None, :]
               + e_tag[None, None, :, :])
    et[...] = jnp.dot(wt[...], cw_[21:42], preferred_element_type=jnp.float32)
    eq[...] = jnp.dot(wq[...], cw_[42:63], preferred_element_type=jnp.float32)


def _fold_tables(wi, wt, wq, wg, wtt, cw):
    return pl.pallas_call(
        _fold_body,
        out_shape=(
            jax.ShapeDtypeStruct((3, 10, NG1, D), jnp.float32),
            jax.ShapeDtypeStruct((NT1, D), jnp.float32),
            jax.ShapeDtypeStruct((NQ1, D), jnp.float32),
        ),
    )(wi, wt, wq, wg, wtt, cw)


def _sc_gather_body(ecomb, etest, eq, idxc2, test2, q2, g_out,
                    ic, it, iq, acc, gb, gc, acc2, sem):
    wid = lax.axis_index("s") * 2 + lax.axis_index("c")
    row0 = wid * (ROWS_PER_W // CHUNK)

    def outer(o, carry_o):
        ro = row0 + o * N_INNER
        pltpu.sync_copy(idxc2.at[pl.ds(ro, N_INNER)], ic)
        pltpu.sync_copy(test2.at[pl.ds(ro, N_INNER)], it)
        pltpu.sync_copy(q2.at[pl.ds(ro, N_INNER)], iq)

        def inner(j, carry_i):
            r0 = (ro + j) * CHUNK
            c1 = pltpu.async_copy(ecomb.at[ic.at[j]], acc, sem)
            c2 = pltpu.async_copy(etest.at[it.at[j]], gb, sem)
            c3 = pltpu.async_copy(eq.at[iq.at[j]], gc, sem)
            c1.wait()
            c2.wait()
            c3.wait()

            def addrow(rr, carry_r):
                ti = rr // 8
                si = rr % 8
                for cc in range(4):
                    s = pl.ds(cc * 16, 16)
                    acc2[ti, si, s] = acc[rr, s] + gb[rr, s] + gc[rr, s]
                return carry_r

            lax.fori_loop(0, CHUNK, addrow, 0)
            pltpu.sync_copy(acc2,
                            g_out.at[pl.ds(r0 // 8, CHUNK // 8), :,
                                     pl.ds(0, D)])
            return carry_i

        lax.fori_loop(0, N_INNER, inner, 0)
        return carry_o

    lax.fori_loop(0, N_OUTER, outer, 0)


def _sc_gather(ecomb, etest, eq, idxc2, test2, q2):
    mesh = plsc.VectorSubcoreMesh(core_axis_name="c", subcore_axis_name="s")
    run = functools.partial(
        pl.kernel,
        mesh=mesh,
        compiler_params=pltpu.CompilerParams(use_tc_tiling_on_sc=False),
        out_type=jax.ShapeDtypeStruct((R // 8, 8, 128), jnp.float32),
        scratch_types=[
            pltpu.VMEM((N_INNER, CHUNK), jnp.int32),
            pltpu.VMEM((N_INNER, CHUNK), jnp.int32),
            pltpu.VMEM((N_INNER, CHUNK), jnp.int32),
            pltpu.VMEM((CHUNK, D), jnp.float32),
            pltpu.VMEM((CHUNK, D), jnp.float32),
            pltpu.VMEM((CHUNK, D), jnp.float32),
            pltpu.VMEM((CHUNK // 8, 8, D), jnp.float32),
            pltpu.SemaphoreType.DMA,
        ],
    )(_sc_gather_body)
    return run(ecomb, etest, eq, idxc2, test2, q2)


_BB = 16
_BR = _BB * L


def _final_body(f_ref, g_ref, wexp_ref, bexp_ref, csig_ref, bc_ref, x_ref):
    f_blk = f_ref[...]
    t = lax.dot_general(f_blk, wexp_ref[...],
                        (((0,), (0,)), ((), ())),
                        preferred_element_type=jnp.float32)
    sg = jax.nn.sigmoid(t + bexp_ref[...])
    gv = g_ref[...][:, :, 0:D].reshape(_BR, D)
    x = (jnp.dot(sg, csig_ref[...], preferred_element_type=jnp.float32)
         + bc_ref[...] + gv)
    x_ref[...] = x.reshape(_BB, L, D)


def _final(f, g3, wexp, bexp, csig, bc):
    return pl.pallas_call(
        _final_body,
        grid=(B // _BB,),
        in_specs=[
            pl.BlockSpec((12, _BR), lambda i: (0, i)),
            pl.BlockSpec((_BR // 8, 8, 128), lambda i: (i, 0, 0)),
            pl.BlockSpec((12, D), lambda i: (0, 0)),
            pl.BlockSpec((1, D), lambda i: (0, 0)),
            pl.BlockSpec((D, D), lambda i: (0, 0)),
            pl.BlockSpec((1, D), lambda i: (0, 0)),
        ],
        out_specs=pl.BlockSpec((_BB, L, D), lambda i: (i, 0, 0)),
        out_shape=jax.ShapeDtypeStruct((B, L, D), jnp.float32),
    )(f, g3, wexp, bexp, csig, bc)


def kernel(test, question, tag, correct, mask, interaction, dffclt, dscrmn,
           gussng, testTag, user_correct_answer, user_total_answer, user_acc,
           user_mean, assessment_mean, test_mean, knowledgeTag_mean,
           time_to_solve, prior_testTag_frequency, W_interaction, W_test,
           W_question, W_tag, W_testTag, lin_w, lin_b, comb_w, comb_b):
    int2 = interaction.reshape(IDX_ROWS, CHUNK)
    tt2 = testTag.reshape(IDX_ROWS, CHUNK)
    tg2 = tag.reshape(IDX_ROWS, CHUNK)
    test2 = test.reshape(IDX_ROWS, CHUNK)
    q2 = question.reshape(IDX_ROWS, CHUNK)

    idxc2 = _compute_idxc(int2, tt2, tg2)
    e4, etest, eq = _fold_tables(W_interaction, W_test, W_question, W_tag,
                                 W_testTag, comb_w)
    ecomb = e4.reshape(NCOMB, D)
    g3 = _sc_gather(ecomb, etest, eq, idxc2, test2, q2)

    segs = [84, 89, 94] + [104 + 5 * i for i in range(9)]
    wexp = jnp.zeros((12, D), jnp.float32)
    bexp = jnp.zeros((1, D), jnp.float32)
    csig = jnp.zeros((D, D), jnp.float32)
    for k in range(12):
        wexp = wexp.at[k, 5 * k:5 * k + 5].set(lin_w)
        bexp = bexp.at[0, 5 * k:5 * k + 5].set(lin_b)
        csig = csig.at[5 * k:5 * k + 5, :].set(comb_w[segs[k]:segs[k] + 5, :])

    f = jnp.stack([dffclt, dscrmn, gussng, user_correct_answer,
                   user_total_answer, user_acc, user_mean, assessment_mean,
                   test_mean, knowledgeTag_mean, time_to_solve,
                   prior_testTag_frequency], axis=0).reshape(12, R)

    return _final(f, g3, wexp, bexp, csig, comb_b.reshape(1, D))


# speedup vs baseline: 14.2841x; 1.1765x over previous
import functools

import jax
import jax.numpy as jnp
from jax import lax
from jax.experimental import pallas as pl
from jax.experimental.pallas import tpu as pltpu
from jax.experimental.pallas import tpu_sc as plsc

B, L = 4096, 200
R = B * L
D = 64
N_TESTS, N_QUESTIONS, N_TAGS = 1538, 9455, 913
NT1, NQ1, NG1 = N_TESTS + 1, N_QUESTIONS + 1, N_TAGS + 1
NCOMB = 3 * 10 * NG1

NW = 32
ROWS_PER_W = R // NW
CHUNK = 128
N_INNER = 8
N_OUTER = ROWS_PER_W // (CHUNK * N_INNER)
IDX_ROWS = R // CHUNK


def _idx_body(it_ref, tt_ref, tg_ref, out_ref):
    out_ref[...] = (it_ref[...] * 10 + tt_ref[...]) * NG1 + tg_ref[...]


def _compute_idxc(int2, tt2, tg2):
    blk = 128
    return pl.pallas_call(
        _idx_body,
        grid=(IDX_ROWS // blk,),
        in_specs=[pl.BlockSpec((blk, CHUNK), lambda i: (i, 0))] * 3,
        out_specs=pl.BlockSpec((blk, CHUNK), lambda i: (i, 0)),
        out_shape=jax.ShapeDtypeStruct((IDX_ROWS, CHUNK), jnp.int32),
    )(int2, tt2, tg2)


def _fold_body(wi, wt, wq, wg, wtt, cw, e4, et, eq):
    cw_ = cw[...]
    e_int = jnp.dot(wi[...], cw_[0:21], preferred_element_type=jnp.float32)
    e_tag = jnp.dot(wg[...], cw_[63:84], preferred_element_type=jnp.float32)
    e_tt = jnp.dot(wtt[...], cw_[99:104], preferred_element_type=jnp.float32)
    e4[...] = (e_int[:, None, None, :] + e_tt[None, :, None, :]
               + e_tag[None, None, :, :])
    et[...] = jnp.dot(wt[...], cw_[21:42], preferred_element_type=jnp.float32)
    eq[...] = jnp.dot(wq[...], cw_[42:63], preferred_element_type=jnp.float32)


def _fold_tables(wi, wt, wq, wg, wtt, cw):
    return pl.pallas_call(
        _fold_body,
        out_shape=(
            jax.ShapeDtypeStruct((3, 10, NG1, D), jnp.float32),
            jax.ShapeDtypeStruct((NT1, D), jnp.float32),
            jax.ShapeDtypeStruct((NQ1, D), jnp.float32),
        ),
    )(wi, wt, wq, wg, wtt, cw)


_HALF = 100


def _sc_gather_body(ecomb, etest, eq, idxc2, test2, q2, g_out,
                    ic, it, iq, aa0, gb0, gc0, aa1, gb1, gc1, o0, o1,
                    gsem0, gsem1, ssem0, ssem1):
    wid = lax.axis_index("s") * 2 + lax.axis_index("c")
    row0 = wid * (ROWS_PER_W // CHUNK)
    aa = (aa0, aa1)
    gb = (gb0, gb1)
    gc = (gc0, gc1)
    oo = (o0, o1)
    gsem = (gsem0, gsem1)
    ssem = (ssem0, ssem1)

    def fire(p, j):
        pltpu.async_copy(ecomb.at[ic.at[j]], aa[p], gsem[p])
        pltpu.async_copy(etest.at[it.at[j]], gb[p], gsem[p])
        pltpu.async_copy(eq.at[iq.at[j]], gc[p], gsem[p])

    def wait_g(p):
        pltpu.make_async_copy(ecomb.at[pl.ds(0, CHUNK)], aa[p], gsem[p]).wait()
        pltpu.make_async_copy(ecomb.at[pl.ds(0, CHUNK)], gb[p], gsem[p]).wait()
        pltpu.make_async_copy(ecomb.at[pl.ds(0, CHUNK)], gc[p], gsem[p]).wait()

    def store(p, r0):
        pltpu.async_copy(
            oo[p], g_out.at[pl.ds(r0 // 8, CHUNK // 8), :, pl.ds(0, D)],
            ssem[p])

    def wait_s(p):
        pltpu.make_async_copy(
            oo[p], g_out.at[pl.ds(0, CHUNK // 8), :, pl.ds(0, D)],
            ssem[p]).wait()

    for h in range(2):
        hrow = row0 + h * _HALF
        pltpu.sync_copy(idxc2.at[pl.ds(hrow, _HALF)], ic)
        pltpu.sync_copy(test2.at[pl.ds(hrow, _HALF)], it)
        pltpu.sync_copy(q2.at[pl.ds(hrow, _HALF)], iq)
        fire(0, 0)

        def pair(i, carry):
            for p in (0, 1):
                c = 2 * i + p

                @pl.when(c < _HALF - 1)
                def _():
                    fire(1 - p, c + 1)

                wait_g(p)

                @pl.when(c >= 2)
                def _():
                    wait_s(p)

                av, bv, cv, ov = aa[p], gb[p], gc[p], oo[p]

                def addrow(rr, carry_r):
                    ti = rr // 8
                    si = rr % 8
                    for cc in range(4):
                        s = pl.ds(cc * 16, 16)
                        ov[ti, si, s] = av[rr, s] + bv[rr, s] + cv[rr, s]
                    return carry_r

                lax.fori_loop(0, CHUNK, addrow, 0)

                store(p, (hrow + c) * CHUNK)
            return carry

        lax.fori_loop(0, _HALF // 2, pair, 0)
        wait_s(0)
        wait_s(1)


def _sc_gather(ecomb, etest, eq, idxc2, test2, q2):
    mesh = plsc.VectorSubcoreMesh(core_axis_name="c", subcore_axis_name="s")
    run = functools.partial(
        pl.kernel,
        mesh=mesh,
        compiler_params=pltpu.CompilerParams(use_tc_tiling_on_sc=False),
        out_type=jax.ShapeDtypeStruct((R // 8, 8, 128), jnp.float32),
        scratch_types=[
            pltpu.VMEM((_HALF, CHUNK), jnp.int32),
            pltpu.VMEM((_HALF, CHUNK), jnp.int32),
            pltpu.VMEM((_HALF, CHUNK), jnp.int32),
            pltpu.VMEM((CHUNK, D), jnp.float32),
            pltpu.VMEM((CHUNK, D), jnp.float32),
            pltpu.VMEM((CHUNK, D), jnp.float32),
            pltpu.VMEM((CHUNK, D), jnp.float32),
            pltpu.VMEM((CHUNK, D), jnp.float32),
            pltpu.VMEM((CHUNK, D), jnp.float32),
            pltpu.VMEM((CHUNK // 8, 8, D), jnp.float32),
            pltpu.VMEM((CHUNK // 8, 8, D), jnp.float32),
            pltpu.SemaphoreType.DMA,
            pltpu.SemaphoreType.DMA,
            pltpu.SemaphoreType.DMA,
            pltpu.SemaphoreType.DMA,
        ],
    )(_sc_gather_body)
    return run(ecomb, etest, eq, idxc2, test2, q2)


_BB = 16
_BR = _BB * L


def _final_body(f_ref, g_ref, wexp_ref, bexp_ref, csig_ref, bc_ref, x_ref):
    f_blk = f_ref[...]
    t = lax.dot_general(f_blk, wexp_ref[...],
                        (((0,), (0,)), ((), ())),
                        preferred_element_type=jnp.float32)
    sg = jax.nn.sigmoid(t + bexp_ref[...])
    gv = g_ref[...][:, :, 0:D].reshape(_BR, D)
    x = (jnp.dot(sg, csig_ref[...], preferred_element_type=jnp.float32)
         + bc_ref[...] + gv)
    x_ref[...] = x.reshape(_BB, L, D)


def _final(f, g3, wexp, bexp, csig, bc):
    return pl.pallas_call(
        _final_body,
        grid=(B // _BB,),
        in_specs=[
            pl.BlockSpec((12, _BR), lambda i: (0, i)),
            pl.BlockSpec((_BR // 8, 8, 128), lambda i: (i, 0, 0)),
            pl.BlockSpec((12, D), lambda i: (0, 0)),
            pl.BlockSpec((1, D), lambda i: (0, 0)),
            pl.BlockSpec((D, D), lambda i: (0, 0)),
            pl.BlockSpec((1, D), lambda i: (0, 0)),
        ],
        out_specs=pl.BlockSpec((_BB, L, D), lambda i: (i, 0, 0)),
        out_shape=jax.ShapeDtypeStruct((B, L, D), jnp.float32),
    )(f, g3, wexp, bexp, csig, bc)


def kernel(test, question, tag, correct, mask, interaction, dffclt, dscrmn,
           gussng, testTag, user_correct_answer, user_total_answer, user_acc,
           user_mean, assessment_mean, test_mean, knowledgeTag_mean,
           time_to_solve, prior_testTag_frequency, W_interaction, W_test,
           W_question, W_tag, W_testTag, lin_w, lin_b, comb_w, comb_b):
    int2 = interaction.reshape(IDX_ROWS, CHUNK)
    tt2 = testTag.reshape(IDX_ROWS, CHUNK)
    tg2 = tag.reshape(IDX_ROWS, CHUNK)
    test2 = test.reshape(IDX_ROWS, CHUNK)
    q2 = question.reshape(IDX_ROWS, CHUNK)

    idxc2 = _compute_idxc(int2, tt2, tg2)
    e4, etest, eq = _fold_tables(W_interaction, W_test, W_question, W_tag,
                                 W_testTag, comb_w)
    ecomb = e4.reshape(NCOMB, D)
    g3 = _sc_gather(ecomb, etest, eq, idxc2, test2, q2)

    segs = [84, 89, 94] + [104 + 5 * i for i in range(9)]
    wexp = jnp.zeros((12, D), jnp.float32)
    bexp = jnp.zeros((1, D), jnp.float32)
    csig = jnp.zeros((D, D), jnp.float32)
    for k in range(12):
        wexp = wexp.at[k, 5 * k:5 * k + 5].set(lin_w)
        bexp = bexp.at[0, 5 * k:5 * k + 5].set(lin_b)
        csig = csig.at[5 * k:5 * k + 5, :].set(comb_w[segs[k]:segs[k] + 5, :])

    f = jnp.stack([dffclt, dscrmn, gussng, user_correct_answer,
                   user_total_answer, user_acc, user_mean, assessment_mean,
                   test_mean, knowledgeTag_mean, time_to_solve,
                   prior_testTag_frequency], axis=0).reshape(12, R)

    return _final(f, g3, wexp, bexp, csig, comb_b.reshape(1, D))
